# baseline (speedup 1.0000x reference)
import jax
import jax.numpy as jnp
from jax import lax
from jax.experimental import pallas as pl
from jax.experimental.pallas import tpu as pltpu
from jax.experimental.pallas import tpu_sc as plsc

N = 8192
D = 1024
S = 2048
L = 16
HC = D // L
HU = 4
NW = 32
TPW = N // NW
C = 8
NCH = TPW // C
NG = NCH // 2
EPS = 1e-12

_GDN = lax.GatherDimensionNumbers(offset_dims=(), collapsed_slice_dims=(0,),
                                  start_index_map=(0,))


def _lanes(v, idx):
    return lax.gather(v, idx[:, None], _GDN, slice_sizes=(1,),
                      mode=lax.GatherScatterMode.PROMISE_IN_BOUNDS)


def _allreduce16(v):
    for shift in (8, 4, 2, 1):
        v = v + _lanes(v, lax.iota(jnp.int32, 16) ^ shift)
    return v


def _rsqrt16(x):
    i = lax.bitcast_convert_type(x, jnp.int32)
    i = jnp.int32(0x5F3759DF) - lax.shift_right_logical(i, 1)
    y = lax.bitcast_convert_type(i, jnp.float32)
    for _ in range(3):
        y = y * (1.5 - 0.5 * x * y * y)
    return y


def _sc_body(ids_ref, tti_ref, sgi_ref, word_ref, pos_ref, tt_ref, seg_ref,
             gam_ref, bet_ref, out_ref,
             idxw, idxt, idxs, wbufs, pbufs, obufs, ttv, sgv, gv, bv,
             sw0, sw1, sp0, sp1, so0, so1):
    cid = lax.axis_index("c")
    sid = lax.axis_index("s")
    wid = sid * 2 + cid
    base = wid * TPW
    s0 = base % S
    sws, sps, sos = (sw0, sw1), (sp0, sp1), (so0, so1)

    pltpu.sync_copy(ids_ref.at[wid], idxw)
    pltpu.sync_copy(tti_ref.at[wid], idxt)
    pltpu.sync_copy(sgi_ref.at[wid], idxs)
    pltpu.sync_copy(tt_ref, ttv)
    pltpu.sync_copy(seg_ref, sgv)
    pltpu.sync_copy(gam_ref, gv)
    pltpu.sync_copy(bet_ref, bv)

    def issue_in(c, k):
        pltpu.async_copy(word_ref.at[idxw.at[pl.ds(c * C, C)]],
                         wbufs.at[k], sws[k])
        pltpu.async_copy(pos_ref.at[pl.ds(s0 + c * C, C)], pbufs.at[k], sps[k])

    def wait_in(k):
        pltpu.make_async_copy(pos_ref.at[pl.ds(0, C)], wbufs.at[k],
                              sws[k]).wait()
        pltpu.make_async_copy(pos_ref.at[pl.ds(0, C)], pbufs.at[k],
                              sps[k]).wait()

    def compute(c, k, g, tv, sv, off):
        wb, pb, ob = wbufs.at[k], pbufs.at[k], obufs.at[k]
        tis = [tv[off + t] for t in range(C)]
        sis = [sv[off + t] for t in range(C)]

        z = jnp.zeros((L,), jnp.float32)

        @plsc.parallel_loop(0, HC, unroll=8, carry=(z,) * (2 * C))
        def h_acc(h, carry):
            acc = list(carry)
            d = pl.ds(h * L, L)
            for t in range(C):
                v = wb[t, d] + pb[t, d] + ttv[tis[t], d] + sgv[sis[t], d]
                wb[t, d] = v
                acc[t] = acc[t] + v
                acc[C + t] = acc[C + t] + v * v
            return tuple(acc)

        acc = h_acc
        s = list(acc[:C])
        q = list(acc[C:])
        for shift in (8, 4, 2, 1):
            idx = lax.iota(jnp.int32, 16) ^ shift
            s = [x + _lanes(x, idx) for x in s]
            q = [x + _lanes(x, idx) for x in q]
        mvs = [x * (1.0 / D) for x in s]
        xs = [jnp.maximum(x * (1.0 / D) - m * m, 0.0) + EPS
              for x, m in zip(q, mvs)]
        iv = [lax.bitcast_convert_type(x, jnp.int32) for x in xs]
        iv = [jnp.int32(0x5F3759DF) - lax.shift_right_logical(i, 1)
              for i in iv]
        rss = [lax.bitcast_convert_type(i, jnp.float32) for i in iv]
        for _ in range(3):
            rss = [y * (1.5 - 0.5 * x * y * y) for x, y in zip(xs, rss)]

        @pl.when(g > 0)
        def _():
            pltpu.make_async_copy(pos_ref.at[pl.ds(0, C)], ob, sos[k]).wait()

        @plsc.parallel_loop(0, HC, unroll=8)
        def h_norm(h):
            d = pl.ds(h * L, L)
            gh = gv[d]
            bh = bv[d]
            for t in range(C):
                ob[t, d] = (wb[t, d] - mvs[t]) * (rss[t] * gh) + bh
        pltpu.async_copy(ob, out_ref.at[pl.ds(base + c * C, C)], sos[k])

    issue_in(0, 0)

    def pair_body(g, _):
        c0 = 2 * g
        tv = idxt[pl.ds(g * 2 * C, 2 * C)]
        sv = idxs[pl.ds(g * 2 * C, 2 * C)]
        issue_in(c0 + 1, 1)
        wait_in(0)
        compute(c0, 0, g, tv, sv, 0)

        @pl.when(g < NG - 1)
        def _():
            issue_in(c0 + 2, 0)

        wait_in(1)
        compute(c0 + 1, 1, g, tv, sv, C)
        return 0

    lax.fori_loop(0, NG, pair_body, 0)

    pltpu.make_async_copy(pos_ref.at[pl.ds(0, C)], obufs.at[0], so0).wait()
    pltpu.make_async_copy(pos_ref.at[pl.ds(0, C)], obufs.at[1], so1).wait()


@jax.jit
def _run(ids3, tti3, sgi3, word_table, pos_table, tt_table, seg_table,
         gamma, beta):
    mesh = plsc.VectorSubcoreMesh(core_axis_name="c", subcore_axis_name="s")
    kfn = pl.kernel(
        _sc_body,
        out_type=jax.ShapeDtypeStruct((N, D), jnp.float32),
        mesh=mesh,
        scratch_types=[
            pltpu.VMEM((TPW,), jnp.int32),
            pltpu.VMEM((TPW,), jnp.int32),
            pltpu.VMEM((TPW,), jnp.int32),
            pltpu.VMEM((2, C, D), jnp.float32),
            pltpu.VMEM((2, C, D), jnp.float32),
            pltpu.VMEM((2, C, D), jnp.float32),
            pltpu.VMEM((2, D), jnp.float32),
            pltpu.VMEM((64, D), jnp.float32),
            pltpu.VMEM((D,), jnp.float32),
            pltpu.VMEM((D,), jnp.float32),
            pltpu.SemaphoreType.DMA,
            pltpu.SemaphoreType.DMA,
            pltpu.SemaphoreType.DMA,
            pltpu.SemaphoreType.DMA,
            pltpu.SemaphoreType.DMA,
            pltpu.SemaphoreType.DMA,
        ],
    )
    return kfn(ids3, tti3, sgi3, word_table, pos_table, tt_table, seg_table,
               gamma, beta)


def kernel(input_ids, token_type_ids, news_segment_ids, word_table, pos_table,
           tt_table, seg_table, gamma, beta):
    B_, S_ = input_ids.shape
    ids3 = input_ids.reshape(NW, TPW).astype(jnp.int32)
    tti3 = token_type_ids.reshape(NW, TPW).astype(jnp.int32)
    sgi3 = news_segment_ids.reshape(NW, TPW).astype(jnp.int32)
    out = _run(ids3, tti3, sgi3, word_table, pos_table, tt_table, seg_table,
               gamma, beta)
    return out.reshape(B_, S_, D)


# speedup vs baseline: 1.2979x; 1.2979x over previous
import jax
import jax.numpy as jnp
from jax import lax
from jax.experimental import pallas as pl
from jax.experimental.pallas import tpu as pltpu
from jax.experimental.pallas import tpu_sc as plsc

N = 8192
D = 1024
S = 2048
L = 16
HC = D // L
HH = HC // 2
NW = 32
TPW = N // NW
C = 8
NCH = TPW // C
NG = NCH // 2
EPS = 1e-12

_GDN = lax.GatherDimensionNumbers(offset_dims=(), collapsed_slice_dims=(0,),
                                  start_index_map=(0,))


def _lanes(v, idx):
    return lax.gather(v, idx[:, None], _GDN, slice_sizes=(1,),
                      mode=lax.GatherScatterMode.PROMISE_IN_BOUNDS)


def _sc_body(ids_ref, tti_ref, sgi_ref, word_ref, pos_ref, tt_ref, seg_ref,
             gam_ref, bet_ref, out_ref,
             idxw, idxt, idxs, wbufs, pbufs, obufs, vbufs, ttv, sgv, gv, bv,
             sw0, sw1, sp0, sp1, so0, so1):
    cid = lax.axis_index("c")
    sid = lax.axis_index("s")
    wid = sid * 2 + cid
    base = wid * TPW
    s0 = base % S
    sws, sps, sos = (sw0, sw1), (sp0, sp1), (so0, so1)

    pltpu.sync_copy(ids_ref.at[wid], idxw)
    pltpu.sync_copy(tti_ref.at[wid], idxt)
    pltpu.sync_copy(sgi_ref.at[wid], idxs)
    pltpu.sync_copy(tt_ref, ttv)
    pltpu.sync_copy(seg_ref, sgv)
    pltpu.sync_copy(gam_ref, gv)
    pltpu.sync_copy(bet_ref, bv)

    def issue_in(c, k):
        pltpu.async_copy(word_ref.at[idxw.at[pl.ds(c * C, C)]],
                         wbufs.at[k], sws[k])
        pltpu.async_copy(pos_ref.at[pl.ds(s0 + c * C, C)], pbufs.at[k], sps[k])

    def wait_in(k):
        pltpu.make_async_copy(pos_ref.at[pl.ds(0, C)], wbufs.at[k],
                              sws[k]).wait()
        pltpu.make_async_copy(pos_ref.at[pl.ds(0, C)], pbufs.at[k],
                              sps[k]).wait()

    def compute(c, k, g, tv, sv, off):
        wb, pb = wbufs.at[k], pbufs.at[k]
        ob, vb = obufs.at[k], vbufs.at[k]
        tis = [tv[off + t] for t in range(C)]
        sis = [sv[off + t] for t in range(C)]

        z = jnp.zeros((L,), jnp.float32)

        @plsc.parallel_loop(0, HH, unroll=2, carry=(z,) * (2 * C))
        def h_acc(p, carry):
            acc = list(carry)
            d0 = pl.ds(p * 2 * L, L)
            d1 = pl.ds(p * 2 * L + L, L)
            d32 = pl.ds(p * 2 * L, 2 * L)
            for t in range(C):
                pk = sgv[pl.ds(sis[t] * (HH * L) + p * L, L)]
                sa = lax.bitcast_convert_type(
                    lax.shift_left(pk, jnp.int32(16)), jnp.float32)
                sb = lax.bitcast_convert_type(
                    pk & jnp.int32(-65536), jnp.float32)
                v0 = wb[t, d0] + pb[t, d0] + ttv[tis[t], d0] + sa
                v1 = wb[t, d1] + pb[t, d1] + ttv[tis[t], d1] + sb
                ob[t, d0] = v0
                ob[t, d1] = v1
                acc[t] = acc[t] + (v0 + v1)
                acc[C + t] = acc[C + t] + (v0 * v0 + v1 * v1)
            return tuple(acc)

        acc = h_acc
        s = list(acc[:C])
        q = list(acc[C:])
        for shift in (8, 4, 2, 1):
            idx = lax.iota(jnp.int32, 16) ^ shift
            s = [x + _lanes(x, idx) for x in s]
            q = [x + _lanes(x, idx) for x in q]
        mvs = [x * (1.0 / D) for x in s]
        xs = [jnp.maximum(x * (1.0 / D) - m * m, 0.0) + EPS
              for x, m in zip(q, mvs)]
        iv = [lax.bitcast_convert_type(x, jnp.int32) for x in xs]
        iv = [jnp.int32(0x5F3759DF) - lax.shift_right_logical(i, 1)
              for i in iv]
        rss = [lax.bitcast_convert_type(i, jnp.float32) for i in iv]
        for _ in range(3):
            rss = [y * (1.5 - 0.5 * x * y * y) for x, y in zip(xs, rss)]

        @pl.when(g > 0)
        def _():
            pltpu.make_async_copy(pos_ref.at[pl.ds(0, C)], vb, sos[k]).wait()

        @plsc.parallel_loop(0, HC, unroll=4)
        def h_norm(h):
            d = pl.ds(h * L, L)
            gh = gv[d]
            bh = bv[d]
            for t in range(C):
                vb[t, d] = (ob[t, d] - mvs[t]) * (rss[t] * gh) + bh

        pltpu.async_copy(vb, out_ref.at[pl.ds(base + c * C, C)], sos[k])

    issue_in(0, 0)

    def pair_body(g, _):
        c0 = 2 * g
        tv = idxt[pl.ds(g * 2 * C, 2 * C)]
        sv = idxs[pl.ds(g * 2 * C, 2 * C)]
        issue_in(c0 + 1, 1)
        wait_in(0)
        compute(c0, 0, g, tv, sv, 0)

        @pl.when(g < NG - 1)
        def _():
            issue_in(c0 + 2, 0)

        wait_in(1)
        compute(c0 + 1, 1, g, tv, sv, C)
        return 0

    lax.fori_loop(0, NG, pair_body, 0)

    pltpu.make_async_copy(pos_ref.at[pl.ds(0, C)], vbufs.at[0], so0).wait()
    pltpu.make_async_copy(pos_ref.at[pl.ds(0, C)], vbufs.at[1], so1).wait()


@jax.jit
def _run(ids3, tti3, sgi3, word_table, pos_table, tt_table, seg_arr,
         gamma, beta):
    mesh = plsc.VectorSubcoreMesh(core_axis_name="c", subcore_axis_name="s")
    kfn = pl.kernel(
        _sc_body,
        out_type=jax.ShapeDtypeStruct((N, D), jnp.float32),
        mesh=mesh,
        scratch_types=[
            pltpu.VMEM((TPW,), jnp.int32),
            pltpu.VMEM((TPW,), jnp.int32),
            pltpu.VMEM((TPW,), jnp.int32),
            pltpu.VMEM((2, C, D), jnp.float32),
            pltpu.VMEM((2, C, D), jnp.float32),
            pltpu.VMEM((2, C, D), jnp.float32),
            pltpu.VMEM((2, C, D), jnp.float32),
            pltpu.VMEM((2, D), jnp.float32),
            pltpu.VMEM((64 * HH * L,), jnp.int32),
            pltpu.VMEM((D,), jnp.float32),
            pltpu.VMEM((D,), jnp.float32),
            pltpu.SemaphoreType.DMA,
            pltpu.SemaphoreType.DMA,
            pltpu.SemaphoreType.DMA,
            pltpu.SemaphoreType.DMA,
            pltpu.SemaphoreType.DMA,
            pltpu.SemaphoreType.DMA,
        ],
    )
    return kfn(ids3, tti3, sgi3, word_table, pos_table, tt_table, seg_arr,
               gamma, beta)


def kernel(input_ids, token_type_ids, news_segment_ids, word_table, pos_table,
           tt_table, seg_table, gamma, beta):
    B_, S_ = input_ids.shape
    ids3 = input_ids.reshape(NW, TPW).astype(jnp.int32)
    tti3 = token_type_ids.reshape(NW, TPW).astype(jnp.int32)
    sgi3 = news_segment_ids.reshape(NW, TPW).astype(jnp.int32)
    seg_arr = lax.bitcast_convert_type(
        (seg_table.astype(jnp.bfloat16)
         .reshape(-1, HH, 2, L).transpose(0, 1, 3, 2)), jnp.int32).reshape(-1)
    out = _run(ids3, tti3, sgi3, word_table, pos_table, tt_table, seg_arr,
               gamma, beta)
    return out.reshape(B_, S_, D)
